# baseline (speedup 1.0000x reference)
import functools

import jax
import jax.numpy as jnp
from jax import lax
from jax.experimental import pallas as pl
from jax.experimental.pallas import tpu as pltpu
import jax.experimental.pallas.tpu_sc as plsc

_NC = 2
_NS = 16
_NW = _NC * _NS
_L = 16

_H = 128
_DP = 32
_CHUNK = 16

_CODE_SHIFT = 17
_MASK_SHIFT = 18

_KF = 10
_KP = 8


def _sc_logits_kernel(T, D):
    tok_per_w = T // _NW
    n_chunks = tok_per_w // _CHUNK
    rows_per_chunk = _CHUNK * D
    idx_count = tok_per_w * D

    mesh = plsc.VectorSubcoreMesh(
        core_axis_name="c", subcore_axis_name="s",
        num_cores=_NC, num_subcores=_NS)

    @functools.partial(
        pl.kernel,
        out_type=(
            jax.ShapeDtypeStruct((T, _DP), jnp.float32),
            jax.ShapeDtypeStruct((T, _DP), jnp.int32),
        ),
        mesh=mesh,
        compiler_params=pltpu.CompilerParams(needs_layout_passes=False,
                                             use_tc_tiling_on_sc=False),
        scratch_types=[
            pltpu.VMEM((tok_per_w,), jnp.int32),
            pltpu.VMEM((tok_per_w, _DP), jnp.int32),
            pltpu.VMEM((idx_count + _L,), jnp.int32),
            pltpu.VMEM((_CHUNK, _H), jnp.float32),
            pltpu.VMEM((rows_per_chunk, _H), jnp.float32),
            pltpu.VMEM((rows_per_chunk, _H), jnp.float32),
            pltpu.VMEM((_CHUNK, _DP), jnp.float32),
            pltpu.SemaphoreType.DMA,
            pltpu.SemaphoreType.DMA,
            pltpu.SemaphoreType.DMA,
        ],
    )
    def kern(packed_hbm, tgt_hbm, emb_hbm, fc_hbm,
             logits_hbm, prows_hbm,
             tgt_v, prow_v, idx_v, emb_v, rows0_v, rows1_v, lg_v,
             sem, sem0, sem1):
        wid = lax.axis_index("s") * _NC + lax.axis_index("c")
        base = wid * tok_per_w
        lanes = lax.iota(jnp.int32, _L)

        pltpu.sync_copy(tgt_hbm.at[pl.ds(base, tok_per_w)], tgt_v)
        psub = tok_per_w // _KP
        pdescs = [
            pltpu.async_copy(packed_hbm.at[tgt_v.at[pl.ds(j * psub, psub)]],
                             prow_v.at[pl.ds(j * psub, psub)], sem)
            for j in range(_KP)
        ]
        for dsc in pdescs:
            dsc.wait()
        pltpu.sync_copy(prow_v, prows_hbm.at[pl.ds(base, tok_per_w)])

        idx_mask = (1 << _CODE_SHIFT) - 1

        def build_idx(t, _):
            r0 = plsc.load_gather(prow_v, [jnp.full((_L,), t, jnp.int32),
                                           lanes])
            idx_v[pl.ds(t * D, _L)] = r0 & idx_mask
            r1 = plsc.load_gather(prow_v, [jnp.full((_L,), t, jnp.int32),
                                           lanes + _L])
            idx_v[pl.ds(t * D + _L, _L)] = r1 & idx_mask
            return 0

        lax.fori_loop(0, tok_per_w, build_idx, 0)

        lane_rows = lanes * D
        sub = rows_per_chunk // _KF

        def fire(c, rows_b, sem_b):
            for j in range(_KF):
                pltpu.async_copy(
                    fc_hbm.at[idx_v.at[pl.ds(c * rows_per_chunk + j * sub,
                                             sub)]],
                    rows_b.at[pl.ds(j * sub, sub)], sem_b)

        def drain(c, rows_b, sem_b):
            for j in range(_KF):
                pltpu.make_async_copy(
                    fc_hbm.at[idx_v.at[pl.ds(c * rows_per_chunk + j * sub,
                                             sub)]],
                    rows_b.at[pl.ds(j * sub, sub)], sem_b).wait()

        def step(c, rows_b, sem_b, rows_n, sem_n):
            @pl.when(c + 1 < n_chunks)
            def _():
                fire(c + 1, rows_n, sem_n)
            drain(c, rows_b, sem_b)
            tok0 = c * _CHUNK
            pltpu.sync_copy(emb_hbm.at[pl.ds(base + tok0, _CHUNK)], emb_v)

            def do_h(h, accs):
                hv = jnp.full((_L,), h, jnp.int32)
                e = plsc.load_gather(emb_v, [lanes, hv])
                new = []
                for d in range(D):
                    w = plsc.load_gather(rows_b, [lane_rows + d, hv])
                    new.append(accs[d] + w * e)
                return tuple(new)

            zeros = jnp.zeros((_L,), jnp.float32)
            accs = lax.fori_loop(0, _H, do_h, (zeros,) * D)

            for i in range(_CHUNK):
                for j in range(_DP // _L):
                    lg_v[i, pl.ds(j * _L, _L)] = zeros
            for d in range(D):
                plsc.store_scatter(lg_v, [lanes, jnp.full((_L,), d, jnp.int32)],
                                   accs[d])
            pltpu.sync_copy(lg_v, logits_hbm.at[pl.ds(base + tok0, _CHUNK)])

        fire(0, rows0_v, sem0)

        def do_chunk(c, _):
            even = (c % 2) == 0

            @pl.when(even)
            def _():
                step(c, rows0_v, sem0, rows1_v, sem1)

            @pl.when(jnp.logical_not(even))
            def _():
                step(c, rows1_v, sem1, rows0_v, sem0)

            return 0

        lax.fori_loop(0, n_chunks, do_chunk, 0)

    return kern


def _tc_bce_kernel(l_ref, p_ref, s_ref, n_ref):
    logits = l_ref[...]
    p = p_ref[...]
    code = ((p >> _CODE_SHIFT) & 1).astype(jnp.float32)
    m = ((p >> _MASK_SHIFT) & 1).astype(jnp.float32)
    el = (jnp.maximum(logits, 0.0) - logits * code
          + jnp.log1p(jnp.exp(-jnp.abs(logits))))
    s_ref[...] = jnp.sum(el * m).reshape(1, 1)
    n_ref[...] = jnp.sum(m).reshape(1, 1)


def kernel(embedding, target, fc, path_idx, path_codes, path_mask):
    emb = embedding.reshape(-1, embedding.shape[-1])
    tgt = target.reshape(-1).astype(jnp.int32)
    T = emb.shape[0]
    D = path_idx.shape[1]

    packed = (path_idx.astype(jnp.int32)
              | (path_codes.astype(jnp.int32) << _CODE_SHIFT)
              | (path_mask.astype(jnp.int32) << _MASK_SHIFT))
    packed = jnp.pad(packed, ((0, 0), (0, _DP - D)))

    logits, prows = _sc_logits_kernel(T, D)(packed, tgt, emb, fc)

    lr = logits.reshape(T * _DP // _H, _H)
    pr = prows.reshape(T * _DP // _H, _H)
    s, n = pl.pallas_call(
        _tc_bce_kernel,
        out_shape=(jax.ShapeDtypeStruct((1, 1), jnp.float32),
                   jax.ShapeDtypeStruct((1, 1), jnp.float32)),
    )(lr, pr)
    return s[0, 0] / n[0, 0]


# speedup vs baseline: 3.4966x; 3.4966x over previous
import functools

import jax
import jax.numpy as jnp
from jax import lax
from jax.experimental import pallas as pl
from jax.experimental.pallas import tpu as pltpu
import jax.experimental.pallas.tpu_sc as plsc

_NC = 2
_NS = 16
_NW = _NC * _NS
_L = 16

_H = 128
_DP = 32
_CHUNK = 16

_CODE_SHIFT = 17
_MASK_SHIFT = 18

_KF = 10
_KP = 8


def _sc_logits_kernel(T, D):
    tok_per_w = T // _NW
    n_chunks = tok_per_w // _CHUNK
    rows_per_chunk = _CHUNK * D
    idx_count = tok_per_w * D

    mesh = plsc.VectorSubcoreMesh(
        core_axis_name="c", subcore_axis_name="s",
        num_cores=_NC, num_subcores=_NS)

    @functools.partial(
        pl.kernel,
        out_type=(
            jax.ShapeDtypeStruct((T, _DP), jnp.float32),
            jax.ShapeDtypeStruct((T, _DP), jnp.int32),
        ),
        mesh=mesh,
        compiler_params=pltpu.CompilerParams(needs_layout_passes=False,
                                             use_tc_tiling_on_sc=False),
        scratch_types=[
            pltpu.VMEM((tok_per_w,), jnp.int32),
            pltpu.VMEM((tok_per_w, _DP), jnp.int32),
            pltpu.VMEM((idx_count + _L,), jnp.int32),
            pltpu.VMEM((_CHUNK, _H), jnp.float32),
            pltpu.VMEM((rows_per_chunk, _H), jnp.float32),
            pltpu.VMEM((rows_per_chunk, _H), jnp.float32),
            pltpu.VMEM((_CHUNK, _DP), jnp.float32),
            pltpu.SemaphoreType.DMA,
            pltpu.SemaphoreType.DMA,
            pltpu.SemaphoreType.DMA,
        ],
    )
    def kern(packed_hbm, tgt_hbm, emb_hbm, fc_hbm,
             logits_hbm, prows_hbm,
             tgt_v, prow_v, idx_v, emb_v, rows0_v, rows1_v, lg_v,
             sem, sem0, sem1):
        wid = lax.axis_index("s") * _NC + lax.axis_index("c")
        base = wid * tok_per_w
        lanes = lax.iota(jnp.int32, _L)

        pltpu.sync_copy(tgt_hbm.at[pl.ds(base, tok_per_w)], tgt_v)
        psub = tok_per_w // _KP
        pdescs = [
            pltpu.async_copy(packed_hbm.at[tgt_v.at[pl.ds(j * psub, psub)]],
                             prow_v.at[pl.ds(j * psub, psub)], sem)
            for j in range(_KP)
        ]
        for dsc in pdescs:
            dsc.wait()
        pltpu.sync_copy(prow_v, prows_hbm.at[pl.ds(base, tok_per_w)])

        idx_mask = (1 << _CODE_SHIFT) - 1

        def build_idx(t, _):
            tv = jnp.full((_L,), t, jnp.int32)
            spread0 = ((base + t) * _DP + lanes) & 0xFFFF
            r0 = plsc.load_gather(prow_v, [tv, lanes])
            live0 = (r0 >> _MASK_SHIFT) & 1
            idx_v[pl.ds(t * D, _L)] = jnp.where(live0 == 1, r0 & idx_mask,
                                                spread0)
            r1 = plsc.load_gather(prow_v, [tv, lanes + _L])
            live1 = (r1 >> _MASK_SHIFT) & 1
            idx_v[pl.ds(t * D + _L, _L)] = jnp.where(
                live1 == 1, r1 & idx_mask, spread0 + _L)
            return 0

        lax.fori_loop(0, tok_per_w, build_idx, 0)

        lane_rows = lanes * D
        sub = rows_per_chunk // _KF

        def fire(c, rows_b, sem_b):
            for j in range(_KF):
                pltpu.async_copy(
                    fc_hbm.at[idx_v.at[pl.ds(c * rows_per_chunk + j * sub,
                                             sub)]],
                    rows_b.at[pl.ds(j * sub, sub)], sem_b)

        def drain(c, rows_b, sem_b):
            for j in range(_KF):
                pltpu.make_async_copy(
                    fc_hbm.at[idx_v.at[pl.ds(c * rows_per_chunk + j * sub,
                                             sub)]],
                    rows_b.at[pl.ds(j * sub, sub)], sem_b).wait()

        def step(c, rows_b, sem_b, rows_n, sem_n):
            @pl.when(c + 1 < n_chunks)
            def _():
                fire(c + 1, rows_n, sem_n)
            drain(c, rows_b, sem_b)
            tok0 = c * _CHUNK
            pltpu.sync_copy(emb_hbm.at[pl.ds(base + tok0, _CHUNK)], emb_v)

            def do_h(h, accs):
                hv = jnp.full((_L,), h, jnp.int32)
                e = plsc.load_gather(emb_v, [lanes, hv])
                new = []
                for d in range(D):
                    w = plsc.load_gather(rows_b, [lane_rows + d, hv])
                    new.append(accs[d] + w * e)
                return tuple(new)

            zeros = jnp.zeros((_L,), jnp.float32)
            accs = lax.fori_loop(0, _H, do_h, (zeros,) * D)

            for i in range(_CHUNK):
                for j in range(_DP // _L):
                    lg_v[i, pl.ds(j * _L, _L)] = zeros
            for d in range(D):
                plsc.store_scatter(lg_v, [lanes, jnp.full((_L,), d, jnp.int32)],
                                   accs[d])
            pltpu.sync_copy(lg_v, logits_hbm.at[pl.ds(base + tok0, _CHUNK)])

        fire(0, rows0_v, sem0)

        def do_chunk(c, _):
            even = (c % 2) == 0

            @pl.when(even)
            def _():
                step(c, rows0_v, sem0, rows1_v, sem1)

            @pl.when(jnp.logical_not(even))
            def _():
                step(c, rows1_v, sem1, rows0_v, sem0)

            return 0

        lax.fori_loop(0, n_chunks, do_chunk, 0)

    return kern


def _tc_bce_kernel(l_ref, p_ref, s_ref, n_ref):
    logits = l_ref[...]
    p = p_ref[...]
    code = ((p >> _CODE_SHIFT) & 1).astype(jnp.float32)
    m = ((p >> _MASK_SHIFT) & 1).astype(jnp.float32)
    el = (jnp.maximum(logits, 0.0) - logits * code
          + jnp.log1p(jnp.exp(-jnp.abs(logits))))
    s_ref[...] = jnp.sum(el * m).reshape(1, 1)
    n_ref[...] = jnp.sum(m).reshape(1, 1)


def kernel(embedding, target, fc, path_idx, path_codes, path_mask):
    emb = embedding.reshape(-1, embedding.shape[-1])
    tgt = target.reshape(-1).astype(jnp.int32)
    T = emb.shape[0]
    D = path_idx.shape[1]

    packed = (path_idx.astype(jnp.int32)
              | (path_codes.astype(jnp.int32) << _CODE_SHIFT)
              | (path_mask.astype(jnp.int32) << _MASK_SHIFT))
    packed = jnp.pad(packed, ((0, 0), (0, _DP - D)))

    logits, prows = _sc_logits_kernel(T, D)(packed, tgt, emb, fc)

    lr = logits.reshape(T * _DP // _H, _H)
    pr = prows.reshape(T * _DP // _H, _H)
    s, n = pl.pallas_call(
        _tc_bce_kernel,
        out_shape=(jax.ShapeDtypeStruct((1, 1), jnp.float32),
                   jax.ShapeDtypeStruct((1, 1), jnp.float32)),
    )(lr, pr)
    return s[0, 0] / n[0, 0]


# speedup vs baseline: 3.5924x; 1.0274x over previous
import functools

import jax
import jax.numpy as jnp
from jax import lax
from jax.experimental import pallas as pl
from jax.experimental.pallas import tpu as pltpu
import jax.experimental.pallas.tpu_sc as plsc

_NC = 2
_NS = 16
_NW = _NC * _NS
_L = 16

_H = 128
_DP = 32
_CHUNK = 16

_CODE_SHIFT = 17
_MASK_SHIFT = 18

_KF = 10
_KP = 4
_NH = 32
_PSEC = 128


def _sc_logits_kernel(T, D):
    tok_per_w = T // _NW
    n_chunks = tok_per_w // _CHUNK
    rows_per_chunk = _CHUNK * D
    idx_count = tok_per_w * D

    mesh = plsc.VectorSubcoreMesh(
        core_axis_name="c", subcore_axis_name="s",
        num_cores=_NC, num_subcores=_NS)

    @functools.partial(
        pl.kernel,
        out_type=(
            jax.ShapeDtypeStruct((T, _DP), jnp.float32),
            jax.ShapeDtypeStruct((T, _DP), jnp.int32),
        ),
        mesh=mesh,
        compiler_params=pltpu.CompilerParams(needs_layout_passes=False,
                                             use_tc_tiling_on_sc=False),
        scratch_types=[
            pltpu.VMEM((tok_per_w,), jnp.int32),
            pltpu.VMEM((_PSEC, _DP), jnp.int32),
            pltpu.VMEM((idx_count + _L,), jnp.int32),
            pltpu.VMEM((idx_count + (_DP - D + 1) * _L,), jnp.int32),
            pltpu.VMEM((_CHUNK, _H), jnp.float32),
            pltpu.VMEM((2 * rows_per_chunk + _NH, _H), jnp.float32),
            pltpu.VMEM((_CHUNK, _DP), jnp.float32),
            pltpu.SemaphoreType.DMA,
            pltpu.SemaphoreType.DMA,
            pltpu.SemaphoreType.DMA,
        ],
    )
    def kern(packed_hbm, tgt_hbm, emb_hbm, fc_hbm,
             logits_hbm, prows_hbm,
             tgt_v, prow_v, idx_v, rowno_v, emb_v, rows_v, lg_v,
             sem, sem0, sem1):
        wid = lax.axis_index("s") * _NC + lax.axis_index("c")
        base = wid * tok_per_w
        lanes = lax.iota(jnp.int32, _L)
        hot_base = fc_hbm.shape[0] - _NH
        hot_slot = 2 * rows_per_chunk

        pltpu.sync_copy(tgt_hbm.at[pl.ds(base, tok_per_w)], tgt_v)
        pltpu.sync_copy(fc_hbm.at[pl.ds(hot_base, _NH)],
                        rows_v.at[pl.ds(hot_slot, _NH)])

        idx_maskc = (1 << _CODE_SHIFT) - 1
        psub = _PSEC // _KP

        for sec in range(tok_per_w // _PSEC):
            t0 = sec * _PSEC
            pdescs = [
                pltpu.async_copy(
                    packed_hbm.at[tgt_v.at[pl.ds(t0 + j * psub, psub)]],
                    prow_v.at[pl.ds(j * psub, psub)], sem)
                for j in range(_KP)
            ]
            for dsc in pdescs:
                dsc.wait()
            pltpu.sync_copy(prow_v, prows_hbm.at[pl.ds(base + t0, _PSEC)])

            def build_idx(ts, _):
                t = t0 + ts
                chunk = t // _CHUNK
                tin = t % _CHUNK
                par = chunk % 2
                slot_base = par * rows_per_chunk + tin * D
                rpos_base = chunk * rows_per_chunk + tin
                tv = jnp.full((_L,), ts, jnp.int32)
                spread = ((base + t) * _DP + lanes) & 0xFFFF
                for half in range(2):
                    dvec = lanes + half * _L
                    r = plsc.load_gather(prow_v, [tv, dvec])
                    raw = r & idx_maskc
                    live = ((r >> _MASK_SHIFT) & 1) == 1
                    cold = jnp.logical_and(live, raw < hot_base)
                    idx_v[pl.ds(t * D + half * _L, _L)] = jnp.where(
                        cold, raw, spread + half * _L)
                    rno = jnp.where(
                        jnp.logical_and(live, raw >= hot_base),
                        hot_slot + (raw - hot_base),
                        slot_base + dvec)
                    plsc.store_scatter(rowno_v, [rpos_base + dvec * _L], rno)
                return 0

            lax.fori_loop(0, _PSEC, build_idx, 0)

        sub = rows_per_chunk // _KF

        def fire(c, off, sem_b):
            for j in range(_KF):
                pltpu.async_copy(
                    fc_hbm.at[idx_v.at[pl.ds(c * rows_per_chunk + j * sub,
                                             sub)]],
                    rows_v.at[pl.ds(off + j * sub, sub)], sem_b)

        def drain(c, off, sem_b):
            for j in range(_KF):
                pltpu.make_async_copy(
                    fc_hbm.at[idx_v.at[pl.ds(c * rows_per_chunk + j * sub,
                                             sub)]],
                    rows_v.at[pl.ds(off + j * sub, sub)], sem_b).wait()

        def step(c, par, sem_b, sem_n):
            @pl.when(c + 1 < n_chunks)
            def _():
                fire(c + 1, (1 - par) * rows_per_chunk, sem_n)
            drain(c, par * rows_per_chunk, sem_b)
            tok0 = c * _CHUNK
            pltpu.sync_copy(emb_hbm.at[pl.ds(base + tok0, _CHUNK)], emb_v)
            rnos = [rowno_v[pl.ds(c * rows_per_chunk + d * _L, _L)]
                    for d in range(D)]

            def do_h(h, accs):
                hv = jnp.full((_L,), h, jnp.int32)
                e = plsc.load_gather(emb_v, [lanes, hv])
                new = []
                for d in range(D):
                    w = plsc.load_gather(rows_v, [rnos[d], hv])
                    new.append(accs[d] + w * e)
                return tuple(new)

            zeros = jnp.zeros((_L,), jnp.float32)
            accs = lax.fori_loop(0, _H, do_h, (zeros,) * D)

            for i in range(_CHUNK):
                for j in range(_DP // _L):
                    lg_v[i, pl.ds(j * _L, _L)] = zeros
            for d in range(D):
                plsc.store_scatter(lg_v, [lanes, jnp.full((_L,), d, jnp.int32)],
                                   accs[d])
            pltpu.sync_copy(lg_v, logits_hbm.at[pl.ds(base + tok0, _CHUNK)])

        fire(0, 0, sem0)

        def do_chunk(c, _):
            even = (c % 2) == 0

            @pl.when(even)
            def _():
                step(c, 0, sem0, sem1)

            @pl.when(jnp.logical_not(even))
            def _():
                step(c, 1, sem1, sem0)

            return 0

        lax.fori_loop(0, n_chunks, do_chunk, 0)

    return kern


def _tc_bce_kernel(l_ref, p_ref, s_ref, n_ref):
    logits = l_ref[...]
    p = p_ref[...]
    code = ((p >> _CODE_SHIFT) & 1).astype(jnp.float32)
    m = ((p >> _MASK_SHIFT) & 1).astype(jnp.float32)
    el = (jnp.maximum(logits, 0.0) - logits * code
          + jnp.log1p(jnp.exp(-jnp.abs(logits))))
    s_ref[...] = jnp.sum(el * m).reshape(1, 1)
    n_ref[...] = jnp.sum(m).reshape(1, 1)


def kernel(embedding, target, fc, path_idx, path_codes, path_mask):
    emb = embedding.reshape(-1, embedding.shape[-1])
    tgt = target.reshape(-1).astype(jnp.int32)
    T = emb.shape[0]
    D = path_idx.shape[1]

    packed = (path_idx.astype(jnp.int32)
              | (path_codes.astype(jnp.int32) << _CODE_SHIFT)
              | (path_mask.astype(jnp.int32) << _MASK_SHIFT))
    packed = jnp.pad(packed, ((0, 0), (0, _DP - D)))

    logits, prows = _sc_logits_kernel(T, D)(packed, tgt, emb, fc)

    lr = logits.reshape(T * _DP // _H, _H)
    pr = prows.reshape(T * _DP // _H, _H)
    s, n = pl.pallas_call(
        _tc_bce_kernel,
        out_shape=(jax.ShapeDtypeStruct((1, 1), jnp.float32),
                   jax.ShapeDtypeStruct((1, 1), jnp.float32)),
    )(lr, pr)
    return s[0, 0] / n[0, 0]


# speedup vs baseline: 4.0369x; 1.1237x over previous
import functools

import jax
import jax.numpy as jnp
from jax import lax
from jax.experimental import pallas as pl
from jax.experimental.pallas import tpu as pltpu
import jax.experimental.pallas.tpu_sc as plsc

_NC = 2
_NS = 16
_NW = _NC * _NS
_L = 16

_H = 128
_DP = 32
_CHUNK = 16

_CODE_SHIFT = 17
_MASK_SHIFT = 18

_KF = 10
_KP = 4
_NH = 32
_PSEC = 128


def _sc_logits_kernel(T, D):
    tok_per_w = T // _NW
    n_chunks = tok_per_w // _CHUNK
    rows_per_chunk = _CHUNK * D
    idx_count = tok_per_w * D

    mesh = plsc.VectorSubcoreMesh(
        core_axis_name="c", subcore_axis_name="s",
        num_cores=_NC, num_subcores=_NS)

    @functools.partial(
        pl.kernel,
        out_type=(
            jax.ShapeDtypeStruct((T, _DP), jnp.float32),
            jax.ShapeDtypeStruct((T, _DP), jnp.int32),
        ),
        mesh=mesh,
        compiler_params=pltpu.CompilerParams(needs_layout_passes=False,
                                             use_tc_tiling_on_sc=False),
        scratch_types=[
            pltpu.VMEM((tok_per_w,), jnp.int32),
            pltpu.VMEM((_PSEC, _DP), jnp.int32),
            pltpu.VMEM((idx_count + _L,), jnp.int32),
            pltpu.VMEM((idx_count + (_DP - D + 1) * _L,), jnp.int32),
            pltpu.VMEM((_CHUNK, _H), jnp.float32),
            pltpu.VMEM((2 * rows_per_chunk + _NH, _H), jnp.float32),
            pltpu.VMEM((_CHUNK, _DP), jnp.float32),
            pltpu.SMEM((n_chunks,), jnp.int32),
            pltpu.SemaphoreType.DMA,
            pltpu.SemaphoreType.DMA,
            pltpu.SemaphoreType.DMA,
        ],
    )
    def kern(packed_hbm, tgt_hbm, emb_hbm, fc_hbm,
             logits_hbm, prows_hbm,
             tgt_v, prow_v, idx_v, rowno_v, emb_v, rows_v, lg_v, mcnt_s,
             sem, sem0, sem1):
        wid = lax.axis_index("s") * _NC + lax.axis_index("c")
        base = wid * tok_per_w
        lanes = lax.iota(jnp.int32, _L)
        hot_base = fc_hbm.shape[0] - _NH
        hot_slot = 2 * rows_per_chunk

        pltpu.sync_copy(tgt_hbm.at[pl.ds(base, tok_per_w)], tgt_v)
        pltpu.sync_copy(fc_hbm.at[pl.ds(hot_base, _NH)],
                        rows_v.at[pl.ds(hot_slot, _NH)])

        idx_maskc = (1 << _CODE_SHIFT) - 1
        psub = _PSEC // _KP

        def init_idx(k, _):
            idx_v[pl.ds(k * _L, _L)] = ((base * _DP + k * _L) + lanes) & 0xFFFF
            return 0

        lax.fori_loop(0, idx_count // _L, init_idx, 0)

        for sec in range(tok_per_w // _PSEC):
            t0 = sec * _PSEC
            pdescs = [
                pltpu.async_copy(
                    packed_hbm.at[tgt_v.at[pl.ds(t0 + j * psub, psub)]],
                    prow_v.at[pl.ds(j * psub, psub)], sem)
                for j in range(_KP)
            ]
            for dsc in pdescs:
                dsc.wait()
            pltpu.sync_copy(prow_v, prows_hbm.at[pl.ds(base + t0, _PSEC)])

            def build_idx(ts, cnt):
                t = t0 + ts
                chunk = t // _CHUNK
                tin = t % _CHUNK
                par = chunk % 2
                rpos_base = chunk * rows_per_chunk + tin
                tv = jnp.full((_L,), ts, jnp.int32)
                for half in range(2):
                    dvec = lanes + half * _L
                    r = plsc.load_gather(prow_v, [tv, dvec])
                    raw = r & idx_maskc
                    live = ((r >> _MASK_SHIFT) & 1) == 1
                    cold = jnp.logical_and(live, raw < hot_base)
                    coldi = cold.astype(jnp.int32)
                    plsc.store_compressed(
                        idx_v.at[pl.ds(chunk * rows_per_chunk + cnt, _L)],
                        raw, mask=cold)
                    prefix = plsc.cumsum(coldi) - coldi
                    rno = jnp.where(
                        cold, par * rows_per_chunk + cnt + prefix,
                        jnp.where(live, hot_slot + (raw - hot_base), 0))
                    plsc.store_scatter(rowno_v, [rpos_base + dvec * _L], rno)
                    cnt = cnt + jnp.max(
                        plsc.all_reduce_population_count(cold))
                @pl.when(tin == _CHUNK - 1)
                def _():
                    mcnt_s[chunk] = cnt

                return jnp.where(tin == _CHUNK - 1, 0, cnt)

            lax.fori_loop(0, _PSEC, build_idx, jnp.int32(0))

        sub = rows_per_chunk // _KF

        def fire(c, off, sem_b):
            m = mcnt_s[c]
            for j in range(_KF):
                @pl.when(j * sub < m)
                def _():
                    pltpu.async_copy(
                        fc_hbm.at[idx_v.at[pl.ds(c * rows_per_chunk + j * sub,
                                                 sub)]],
                        rows_v.at[pl.ds(off + j * sub, sub)], sem_b)

        def drain(c, off, sem_b):
            m = mcnt_s[c]
            for j in range(_KF):
                @pl.when(j * sub < m)
                def _():
                    pltpu.make_async_copy(
                        fc_hbm.at[idx_v.at[pl.ds(c * rows_per_chunk + j * sub,
                                                 sub)]],
                        rows_v.at[pl.ds(off + j * sub, sub)], sem_b).wait()

        def step(c, par, sem_b, sem_n):
            @pl.when(c + 1 < n_chunks)
            def _():
                fire(c + 1, (1 - par) * rows_per_chunk, sem_n)
            drain(c, par * rows_per_chunk, sem_b)
            tok0 = c * _CHUNK
            pltpu.sync_copy(emb_hbm.at[pl.ds(base + tok0, _CHUNK)], emb_v)
            rnos = [rowno_v[pl.ds(c * rows_per_chunk + d * _L, _L)]
                    for d in range(D)]

            def do_h(h, accs):
                hv = jnp.full((_L,), h, jnp.int32)
                e = plsc.load_gather(emb_v, [lanes, hv])
                new = []
                for d in range(D):
                    w = plsc.load_gather(rows_v, [rnos[d], hv])
                    new.append(accs[d] + w * e)
                return tuple(new)

            zeros = jnp.zeros((_L,), jnp.float32)
            accs = lax.fori_loop(0, _H, do_h, (zeros,) * D)

            for i in range(_CHUNK):
                for j in range(_DP // _L):
                    lg_v[i, pl.ds(j * _L, _L)] = zeros
            for d in range(D):
                plsc.store_scatter(lg_v, [lanes, jnp.full((_L,), d, jnp.int32)],
                                   accs[d])
            pltpu.sync_copy(lg_v, logits_hbm.at[pl.ds(base + tok0, _CHUNK)])

        fire(0, 0, sem0)

        def do_chunk(c, _):
            even = (c % 2) == 0

            @pl.when(even)
            def _():
                step(c, 0, sem0, sem1)

            @pl.when(jnp.logical_not(even))
            def _():
                step(c, 1, sem1, sem0)

            return 0

        lax.fori_loop(0, n_chunks, do_chunk, 0)

    return kern


def _tc_bce_kernel(l_ref, p_ref, s_ref, n_ref):
    logits = l_ref[...]
    p = p_ref[...]
    code = ((p >> _CODE_SHIFT) & 1).astype(jnp.float32)
    m = ((p >> _MASK_SHIFT) & 1).astype(jnp.float32)
    el = (jnp.maximum(logits, 0.0) - logits * code
          + jnp.log1p(jnp.exp(-jnp.abs(logits))))
    s_ref[...] = jnp.sum(el * m).reshape(1, 1)
    n_ref[...] = jnp.sum(m).reshape(1, 1)


def kernel(embedding, target, fc, path_idx, path_codes, path_mask):
    emb = embedding.reshape(-1, embedding.shape[-1])
    tgt = target.reshape(-1).astype(jnp.int32)
    T = emb.shape[0]
    D = path_idx.shape[1]

    packed = (path_idx.astype(jnp.int32)
              | (path_codes.astype(jnp.int32) << _CODE_SHIFT)
              | (path_mask.astype(jnp.int32) << _MASK_SHIFT))
    packed = jnp.pad(packed, ((0, 0), (0, _DP - D)))

    logits, prows = _sc_logits_kernel(T, D)(packed, tgt, emb, fc)

    lr = logits.reshape(T * _DP // _H, _H)
    pr = prows.reshape(T * _DP // _H, _H)
    s, n = pl.pallas_call(
        _tc_bce_kernel,
        out_shape=(jax.ShapeDtypeStruct((1, 1), jnp.float32),
                   jax.ShapeDtypeStruct((1, 1), jnp.float32)),
    )(lr, pr)
    return s[0, 0] / n[0, 0]


# speedup vs baseline: 4.0399x; 1.0008x over previous
import functools

import jax
import jax.numpy as jnp
from jax import lax
from jax.experimental import pallas as pl
from jax.experimental.pallas import tpu as pltpu
import jax.experimental.pallas.tpu_sc as plsc

_NC = 2
_NS = 16
_NW = _NC * _NS
_L = 16

_H = 128
_DP = 32
_CHUNK = 16

_CODE_SHIFT = 17
_MASK_SHIFT = 18

_KF = 8
_NH = 64
_KP = 4
_PSEC = 128
_WIN = 384


def _sc_logits_kernel(T, D):
    tok_per_w = T // _NW
    n_chunks = tok_per_w // _CHUNK
    rows_per_chunk = _CHUNK * D
    idx_count = tok_per_w * D

    mesh = plsc.VectorSubcoreMesh(
        core_axis_name="c", subcore_axis_name="s",
        num_cores=_NC, num_subcores=_NS)

    @functools.partial(
        pl.kernel,
        out_type=(
            jax.ShapeDtypeStruct((T, _DP), jnp.float32),
            jax.ShapeDtypeStruct((T, _DP), jnp.int32),
        ),
        mesh=mesh,
        compiler_params=pltpu.CompilerParams(needs_layout_passes=False,
                                             use_tc_tiling_on_sc=False),
        scratch_types=[
            pltpu.VMEM((tok_per_w,), jnp.int32),
            pltpu.VMEM((_PSEC, _DP), jnp.int32),
            pltpu.VMEM((idx_count + _L,), jnp.int32),
            pltpu.VMEM((idx_count + (_DP - D + 1) * _L,), jnp.int32),
            pltpu.VMEM((_CHUNK, _H), jnp.float32),
            pltpu.VMEM((2 * _WIN + _NH, _H), jnp.float32),
            pltpu.VMEM((_CHUNK, _DP), jnp.float32),
            pltpu.SMEM((n_chunks,), jnp.int32),
            pltpu.SemaphoreType.DMA,
            pltpu.SemaphoreType.DMA,
            pltpu.SemaphoreType.DMA,
        ],
    )
    def kern(packed_hbm, tgt_hbm, emb_hbm, fc_hbm,
             logits_hbm, prows_hbm,
             tgt_v, prow_v, idx_v, rowno_v, emb_v, rows_v, lg_v, mcnt_s,
             sem, sem0, sem1):
        wid = lax.axis_index("s") * _NC + lax.axis_index("c")
        base = wid * tok_per_w
        lanes = lax.iota(jnp.int32, _L)
        hot_base = fc_hbm.shape[0] - _NH
        hot_slot = 2 * _WIN

        pltpu.sync_copy(tgt_hbm.at[pl.ds(base, tok_per_w)], tgt_v)
        pltpu.sync_copy(fc_hbm.at[pl.ds(hot_base, _NH)],
                        rows_v.at[pl.ds(hot_slot, _NH)])

        idx_maskc = (1 << _CODE_SHIFT) - 1
        psub = _PSEC // _KP

        def init_idx(k, _):
            idx_v[pl.ds(k * _L, _L)] = ((base * _DP + k * _L) + lanes) & 0xFFFF
            return 0

        lax.fori_loop(0, idx_count // _L, init_idx, 0)

        for sec in range(tok_per_w // _PSEC):
            t0 = sec * _PSEC
            pdescs = [
                pltpu.async_copy(
                    packed_hbm.at[tgt_v.at[pl.ds(t0 + j * psub, psub)]],
                    prow_v.at[pl.ds(j * psub, psub)], sem)
                for j in range(_KP)
            ]
            for dsc in pdescs:
                dsc.wait()
            pltpu.sync_copy(prow_v, prows_hbm.at[pl.ds(base + t0, _PSEC)])

            def build_idx(ts, cnt):
                t = t0 + ts
                chunk = t // _CHUNK
                tin = t % _CHUNK
                par = chunk % 2
                rpos_base = chunk * rows_per_chunk + tin
                tv = jnp.full((_L,), ts, jnp.int32)
                for half in range(2):
                    dvec = lanes + half * _L
                    r = plsc.load_gather(prow_v, [tv, dvec])
                    raw = r & idx_maskc
                    live = ((r >> _MASK_SHIFT) & 1) == 1
                    cold = jnp.logical_and(live, raw < hot_base)
                    coldi = cold.astype(jnp.int32)
                    plsc.store_compressed(
                        idx_v.at[pl.ds(chunk * rows_per_chunk + cnt, _L)],
                        raw, mask=cold)
                    prefix = plsc.cumsum(coldi) - coldi
                    rno = jnp.where(
                        cold, par * _WIN + cnt + prefix,
                        jnp.where(live, hot_slot + (raw - hot_base), 0))
                    plsc.store_scatter(rowno_v, [rpos_base + dvec * _L], rno)
                    cnt = cnt + jnp.max(
                        plsc.all_reduce_population_count(cold))
                @pl.when(tin == _CHUNK - 1)
                def _():
                    mcnt_s[chunk] = cnt

                return jnp.where(tin == _CHUNK - 1, 0, cnt)

            lax.fori_loop(0, _PSEC, build_idx, jnp.int32(0))

        sub = _WIN // _KF

        def fire(c, off, sem_b):
            m = mcnt_s[c]
            for j in range(_KF):
                @pl.when(j * sub < m)
                def _():
                    pltpu.async_copy(
                        fc_hbm.at[idx_v.at[pl.ds(c * rows_per_chunk + j * sub,
                                                 sub)]],
                        rows_v.at[pl.ds(off + j * sub, sub)], sem_b)

        def drain(c, off, sem_b):
            m = mcnt_s[c]
            for j in range(_KF):
                @pl.when(j * sub < m)
                def _():
                    pltpu.make_async_copy(
                        fc_hbm.at[idx_v.at[pl.ds(c * rows_per_chunk + j * sub,
                                                 sub)]],
                        rows_v.at[pl.ds(off + j * sub, sub)], sem_b).wait()

        def step(c, par, sem_b, sem_n):
            @pl.when(c + 1 < n_chunks)
            def _():
                fire(c + 1, (1 - par) * _WIN, sem_n)
            drain(c, par * _WIN, sem_b)
            tok0 = c * _CHUNK
            pltpu.sync_copy(emb_hbm.at[pl.ds(base + tok0, _CHUNK)], emb_v)
            rnos = [rowno_v[pl.ds(c * rows_per_chunk + d * _L, _L)]
                    for d in range(D)]

            def do_h(h, accs):
                hv = jnp.full((_L,), h, jnp.int32)
                e = plsc.load_gather(emb_v, [lanes, hv])
                new = []
                for d in range(D):
                    w = plsc.load_gather(rows_v, [rnos[d], hv])
                    new.append(accs[d] + w * e)
                return tuple(new)

            zeros = jnp.zeros((_L,), jnp.float32)
            accs = lax.fori_loop(0, _H, do_h, (zeros,) * D)

            for i in range(_CHUNK):
                for j in range(_DP // _L):
                    lg_v[i, pl.ds(j * _L, _L)] = zeros
            for d in range(D):
                plsc.store_scatter(lg_v, [lanes, jnp.full((_L,), d, jnp.int32)],
                                   accs[d])
            pltpu.sync_copy(lg_v, logits_hbm.at[pl.ds(base + tok0, _CHUNK)])

        fire(0, 0, sem0)

        def do_chunk(c, _):
            even = (c % 2) == 0

            @pl.when(even)
            def _():
                step(c, 0, sem0, sem1)

            @pl.when(jnp.logical_not(even))
            def _():
                step(c, 1, sem1, sem0)

            return 0

        lax.fori_loop(0, n_chunks, do_chunk, 0)

    return kern


def _tc_bce_kernel(l_ref, p_ref, s_ref, n_ref):
    logits = l_ref[...]
    p = p_ref[...]
    code = ((p >> _CODE_SHIFT) & 1).astype(jnp.float32)
    m = ((p >> _MASK_SHIFT) & 1).astype(jnp.float32)
    el = (jnp.maximum(logits, 0.0) - logits * code
          + jnp.log1p(jnp.exp(-jnp.abs(logits))))
    s_ref[...] = jnp.sum(el * m).reshape(1, 1)
    n_ref[...] = jnp.sum(m).reshape(1, 1)


def kernel(embedding, target, fc, path_idx, path_codes, path_mask):
    emb = embedding.reshape(-1, embedding.shape[-1])
    tgt = target.reshape(-1).astype(jnp.int32)
    T = emb.shape[0]
    D = path_idx.shape[1]

    packed = (path_idx.astype(jnp.int32)
              | (path_codes.astype(jnp.int32) << _CODE_SHIFT)
              | (path_mask.astype(jnp.int32) << _MASK_SHIFT))
    packed = jnp.pad(packed, ((0, 0), (0, _DP - D)))

    logits, prows = _sc_logits_kernel(T, D)(packed, tgt, emb, fc)

    lr = logits.reshape(T * _DP // _H, _H)
    pr = prows.reshape(T * _DP // _H, _H)
    s, n = pl.pallas_call(
        _tc_bce_kernel,
        out_shape=(jax.ShapeDtypeStruct((1, 1), jnp.float32),
                   jax.ShapeDtypeStruct((1, 1), jnp.float32)),
    )(lr, pr)
    return s[0, 0] / n[0, 0]


# speedup vs baseline: 4.0441x; 1.0010x over previous
import functools

import jax
import jax.numpy as jnp
from jax import lax
from jax.experimental import pallas as pl
from jax.experimental.pallas import tpu as pltpu
import jax.experimental.pallas.tpu_sc as plsc

_NC = 2
_NS = 16
_NW = _NC * _NS
_L = 16

_H = 128
_DP = 32
_CHUNK = 16

_CODE_SHIFT = 17
_MASK_SHIFT = 18

_KF = 4
_NH = 64
_KP = 4
_PSEC = 128
_WIN = 384


def _sc_logits_kernel(T, D):
    tok_per_w = T // _NW
    n_chunks = tok_per_w // _CHUNK
    rows_per_chunk = _CHUNK * D
    idx_count = tok_per_w * D

    mesh = plsc.VectorSubcoreMesh(
        core_axis_name="c", subcore_axis_name="s",
        num_cores=_NC, num_subcores=_NS)

    @functools.partial(
        pl.kernel,
        out_type=(
            jax.ShapeDtypeStruct((T, _DP), jnp.float32),
            jax.ShapeDtypeStruct((T, _DP), jnp.int32),
        ),
        mesh=mesh,
        compiler_params=pltpu.CompilerParams(needs_layout_passes=False,
                                             use_tc_tiling_on_sc=False),
        scratch_types=[
            pltpu.VMEM((tok_per_w,), jnp.int32),
            pltpu.VMEM((_PSEC, _DP), jnp.int32),
            pltpu.VMEM((idx_count + _L,), jnp.int32),
            pltpu.VMEM((idx_count + (_DP - D + 1) * _L,), jnp.int32),
            pltpu.VMEM((_CHUNK, _H), jnp.float32),
            pltpu.VMEM((2 * _WIN + _NH, _H), jnp.float32),
            pltpu.VMEM((_CHUNK, _DP), jnp.float32),
            pltpu.SMEM((n_chunks,), jnp.int32),
            pltpu.SemaphoreType.DMA,
            pltpu.SemaphoreType.DMA,
            pltpu.SemaphoreType.DMA,
        ],
    )
    def kern(packed_hbm, tgt_hbm, emb_hbm, fc_hbm,
             logits_hbm, prows_hbm,
             tgt_v, prow_v, idx_v, rowno_v, emb_v, rows_v, lg_v, mcnt_s,
             sem, sem0, sem1):
        wid = lax.axis_index("s") * _NC + lax.axis_index("c")
        base = wid * tok_per_w
        lanes = lax.iota(jnp.int32, _L)
        hot_base = fc_hbm.shape[0] - _NH
        hot_slot = 2 * _WIN

        pltpu.sync_copy(tgt_hbm.at[pl.ds(base, tok_per_w)], tgt_v)
        pltpu.sync_copy(fc_hbm.at[pl.ds(hot_base, _NH)],
                        rows_v.at[pl.ds(hot_slot, _NH)])

        idx_maskc = (1 << _CODE_SHIFT) - 1
        psub = _PSEC // _KP

        def init_idx(k, _):
            idx_v[pl.ds(k * _L, _L)] = ((base * _DP + k * _L) + lanes) & 0xFFFF
            return 0

        lax.fori_loop(0, idx_count // _L, init_idx, 0)

        for sec in range(tok_per_w // _PSEC):
            t0 = sec * _PSEC
            pdescs = [
                pltpu.async_copy(
                    packed_hbm.at[tgt_v.at[pl.ds(t0 + j * psub, psub)]],
                    prow_v.at[pl.ds(j * psub, psub)], sem)
                for j in range(_KP)
            ]
            for dsc in pdescs:
                dsc.wait()
            pltpu.sync_copy(prow_v, prows_hbm.at[pl.ds(base + t0, _PSEC)])

            def build_idx(ts, cnt):
                t = t0 + ts
                chunk = t // _CHUNK
                tin = t % _CHUNK
                par = chunk % 2
                rpos_base = chunk * rows_per_chunk + tin
                tv = jnp.full((_L,), ts, jnp.int32)
                for half in range(2):
                    dvec = lanes + half * _L
                    r = plsc.load_gather(prow_v, [tv, dvec])
                    raw = r & idx_maskc
                    live = ((r >> _MASK_SHIFT) & 1) == 1
                    cold = jnp.logical_and(live, raw < hot_base)
                    coldi = cold.astype(jnp.int32)
                    plsc.store_compressed(
                        idx_v.at[pl.ds(chunk * rows_per_chunk + cnt, _L)],
                        raw, mask=cold)
                    prefix = plsc.cumsum(coldi) - coldi
                    rno = jnp.where(
                        cold, par * _WIN + cnt + prefix,
                        jnp.where(live, hot_slot + (raw - hot_base), 0))
                    plsc.store_scatter(rowno_v, [rpos_base + dvec * _L], rno)
                    cnt = cnt + jnp.max(
                        plsc.all_reduce_population_count(cold))
                @pl.when(tin == _CHUNK - 1)
                def _():
                    mcnt_s[chunk] = cnt

                return jnp.where(tin == _CHUNK - 1, 0, cnt)

            lax.fori_loop(0, _PSEC, build_idx, jnp.int32(0))

        sub = _WIN // _KF

        def fire(c, off, sem_b):
            m = mcnt_s[c]
            for j in range(_KF):
                @pl.when(j * sub < m)
                def _():
                    pltpu.async_copy(
                        fc_hbm.at[idx_v.at[pl.ds(c * rows_per_chunk + j * sub,
                                                 sub)]],
                        rows_v.at[pl.ds(off + j * sub, sub)], sem_b)

        def drain(c, off, sem_b):
            m = mcnt_s[c]
            for j in range(_KF):
                @pl.when(j * sub < m)
                def _():
                    pltpu.make_async_copy(
                        fc_hbm.at[idx_v.at[pl.ds(c * rows_per_chunk + j * sub,
                                                 sub)]],
                        rows_v.at[pl.ds(off + j * sub, sub)], sem_b).wait()

        def step(c, par, sem_b, sem_n):
            @pl.when(c + 1 < n_chunks)
            def _():
                fire(c + 1, (1 - par) * _WIN, sem_n)
            drain(c, par * _WIN, sem_b)
            tok0 = c * _CHUNK
            pltpu.sync_copy(emb_hbm.at[pl.ds(base + tok0, _CHUNK)], emb_v)
            rnos = [rowno_v[pl.ds(c * rows_per_chunk + d * _L, _L)]
                    for d in range(D)]

            def do_h(h, accs):
                hv = jnp.full((_L,), h, jnp.int32)
                e = plsc.load_gather(emb_v, [lanes, hv])
                new = []
                for d in range(D):
                    w = plsc.load_gather(rows_v, [rnos[d], hv])
                    new.append(accs[d] + w * e)
                return tuple(new)

            zeros = jnp.zeros((_L,), jnp.float32)
            accs = lax.fori_loop(0, _H, do_h, (zeros,) * D)

            for i in range(_CHUNK):
                for j in range(_DP // _L):
                    lg_v[i, pl.ds(j * _L, _L)] = zeros
            for d in range(D):
                plsc.store_scatter(lg_v, [lanes, jnp.full((_L,), d, jnp.int32)],
                                   accs[d])
            pltpu.sync_copy(lg_v, logits_hbm.at[pl.ds(base + tok0, _CHUNK)])

        fire(0, 0, sem0)

        def do_chunk(c, _):
            even = (c % 2) == 0

            @pl.when(even)
            def _():
                step(c, 0, sem0, sem1)

            @pl.when(jnp.logical_not(even))
            def _():
                step(c, 1, sem1, sem0)

            return 0

        lax.fori_loop(0, n_chunks, do_chunk, 0)

    return kern


def _tc_bce_kernel(l_ref, p_ref, s_ref, n_ref):
    logits = l_ref[...]
    p = p_ref[...]
    code = ((p >> _CODE_SHIFT) & 1).astype(jnp.float32)
    m = ((p >> _MASK_SHIFT) & 1).astype(jnp.float32)
    el = (jnp.maximum(logits, 0.0) - logits * code
          + jnp.log1p(jnp.exp(-jnp.abs(logits))))
    s_ref[...] = jnp.sum(el * m).reshape(1, 1)
    n_ref[...] = jnp.sum(m).reshape(1, 1)


def kernel(embedding, target, fc, path_idx, path_codes, path_mask):
    emb = embedding.reshape(-1, embedding.shape[-1])
    tgt = target.reshape(-1).astype(jnp.int32)
    T = emb.shape[0]
    D = path_idx.shape[1]

    packed = (path_idx.astype(jnp.int32)
              | (path_codes.astype(jnp.int32) << _CODE_SHIFT)
              | (path_mask.astype(jnp.int32) << _MASK_SHIFT))
    packed = jnp.pad(packed, ((0, 0), (0, _DP - D)))

    logits, prows = _sc_logits_kernel(T, D)(packed, tgt, emb, fc)

    lr = logits.reshape(T * _DP // _H, _H)
    pr = prows.reshape(T * _DP // _H, _H)
    s, n = pl.pallas_call(
        _tc_bce_kernel,
        out_shape=(jax.ShapeDtypeStruct((1, 1), jnp.float32),
                   jax.ShapeDtypeStruct((1, 1), jnp.float32)),
    )(lr, pr)
    return s[0, 0] / n[0, 0]


# speedup vs baseline: 8.4276x; 2.0839x over previous
import functools

import jax
import jax.numpy as jnp
from jax import lax
from jax.experimental import pallas as pl
from jax.experimental.pallas import tpu as pltpu
import jax.experimental.pallas.tpu_sc as plsc

_NC = 2
_NS = 16
_NW = _NC * _NS
_L = 16

_H = 128
_DP = 32
_CHUNK = 16

_CODE_SHIFT = 17
_MASK_SHIFT = 18

_KF = 4
_NH = 64
_KP = 4
_PSEC = 128
_WIN = 384


def _sc_logits_kernel(T, D):
    tok_per_w = T // _NW
    n_chunks = tok_per_w // _CHUNK
    rows_per_chunk = _CHUNK * D
    idx_count = tok_per_w * D

    mesh = plsc.VectorSubcoreMesh(
        core_axis_name="c", subcore_axis_name="s",
        num_cores=_NC, num_subcores=_NS)

    @functools.partial(
        pl.kernel,
        out_type=(
            jax.ShapeDtypeStruct((T, _DP), jnp.float32),
            jax.ShapeDtypeStruct((T, _DP), jnp.int32),
        ),
        mesh=mesh,
        compiler_params=pltpu.CompilerParams(needs_layout_passes=False,
                                             use_tc_tiling_on_sc=False),
        scratch_types=[
            pltpu.VMEM((tok_per_w,), jnp.int32),
            pltpu.VMEM((_PSEC, _DP), jnp.int32),
            pltpu.VMEM((idx_count + _L,), jnp.int32),
            pltpu.VMEM((idx_count + (_DP - D + 1) * _L,), jnp.int32),
            pltpu.VMEM((_CHUNK, _H), jnp.float32),
            pltpu.VMEM((2 * _WIN + _NH, _H), jnp.float32),
            pltpu.VMEM((_CHUNK, _DP), jnp.float32),
            pltpu.SMEM((n_chunks,), jnp.int32),
            pltpu.SemaphoreType.DMA,
            pltpu.SemaphoreType.DMA,
            pltpu.SemaphoreType.DMA,
        ],
    )
    def kern(packed_hbm, tgt_hbm, emb_hbm, fc_hbm,
             logits_hbm, prows_hbm,
             tgt_v, prow_v, idx_v, rowno_v, emb_v, rows_v, lg_v, mcnt_s,
             sem, sem0, sem1):
        wid = lax.axis_index("s") * _NC + lax.axis_index("c")
        base = wid * tok_per_w
        lanes = lax.iota(jnp.int32, _L)
        hot_base = fc_hbm.shape[0] - _NH
        hot_slot = 2 * _WIN

        pltpu.sync_copy(tgt_hbm.at[pl.ds(base, tok_per_w)], tgt_v)
        pltpu.sync_copy(fc_hbm.at[pl.ds(hot_base, _NH)],
                        rows_v.at[pl.ds(hot_slot, _NH)])

        idx_maskc = (1 << _CODE_SHIFT) - 1
        psub = _PSEC // _KP

        def init_idx(k, _):
            idx_v[pl.ds(k * _L, _L)] = ((base * _DP + k * _L) + lanes) & 0xFFFF
            return 0

        lax.fori_loop(0, idx_count // _L, init_idx, 0)

        for sec in range(tok_per_w // _PSEC):
            t0 = sec * _PSEC
            pdescs = [
                pltpu.async_copy(
                    packed_hbm.at[tgt_v.at[pl.ds(t0 + j * psub, psub)]],
                    prow_v.at[pl.ds(j * psub, psub)], sem)
                for j in range(_KP)
            ]
            for dsc in pdescs:
                dsc.wait()
            pltpu.sync_copy(prow_v, prows_hbm.at[pl.ds(base + t0, _PSEC)])

            def build_idx(ts, cnt):
                t = t0 + ts
                chunk = t // _CHUNK
                tin = t % _CHUNK
                par = chunk % 2
                rpos_base = chunk * rows_per_chunk + tin
                tv = jnp.full((_L,), ts, jnp.int32)
                for half in range(2):
                    dvec = lanes + half * _L
                    r = plsc.load_gather(prow_v, [tv, dvec])
                    raw = r & idx_maskc
                    live = ((r >> _MASK_SHIFT) & 1) == 1
                    cold = jnp.logical_and(live, raw < hot_base)
                    coldi = cold.astype(jnp.int32)
                    plsc.store_compressed(
                        idx_v.at[pl.ds(chunk * rows_per_chunk + cnt, _L)],
                        raw, mask=cold)
                    prefix = plsc.cumsum(coldi) - coldi
                    rno = jnp.where(
                        cold, par * _WIN + cnt + prefix,
                        jnp.where(live, hot_slot + (raw - hot_base), 0))
                    plsc.store_scatter(rowno_v, [rpos_base + dvec * _L], rno)
                    cnt = cnt + jnp.max(
                        plsc.all_reduce_population_count(cold))
                @pl.when(tin == _CHUNK - 1)
                def _():
                    mcnt_s[chunk] = cnt

                return jnp.where(tin == _CHUNK - 1, 0, cnt)

            lax.fori_loop(0, _PSEC, build_idx, jnp.int32(0))

        sub = _WIN // _KF

        def fire(c, off, sem_b):
            m = mcnt_s[c]
            for j in range(_KF):
                @pl.when(j * sub < m)
                def _():
                    pltpu.async_copy(
                        fc_hbm.at[idx_v.at[pl.ds(c * rows_per_chunk + j * sub,
                                                 sub)]],
                        rows_v.at[pl.ds(off + j * sub, sub)], sem_b)

        def drain(c, off, sem_b):
            m = mcnt_s[c]
            for j in range(_KF):
                @pl.when(j * sub < m)
                def _():
                    pltpu.make_async_copy(
                        fc_hbm.at[idx_v.at[pl.ds(c * rows_per_chunk + j * sub,
                                                 sub)]],
                        rows_v.at[pl.ds(off + j * sub, sub)], sem_b).wait()

        def step(c, par, sem_b, sem_n):
            @pl.when(c + 1 < n_chunks)
            def _():
                fire(c + 1, (1 - par) * _WIN, sem_n)
            drain(c, par * _WIN, sem_b)
            tok0 = c * _CHUNK
            pltpu.sync_copy(emb_hbm.at[pl.ds(base + tok0, _CHUNK)], emb_v)
            rnos = [rowno_v[pl.ds(c * rows_per_chunk + d * _L, _L)]
                    for d in range(D)]

            def do_h(h, accs):
                cv = (lanes + h) & (_H - 1)
                e = plsc.load_gather(emb_v, [lanes, cv])
                new = []
                for d in range(D):
                    w = plsc.load_gather(rows_v, [rnos[d], cv])
                    new.append(accs[d] + w * e)
                return tuple(new)

            zeros = jnp.zeros((_L,), jnp.float32)
            accs = lax.fori_loop(0, _H, do_h, (zeros,) * D)

            for i in range(_CHUNK):
                for j in range(_DP // _L):
                    lg_v[i, pl.ds(j * _L, _L)] = zeros
            for d in range(D):
                plsc.store_scatter(lg_v, [lanes, jnp.full((_L,), d, jnp.int32)],
                                   accs[d])
            pltpu.sync_copy(lg_v, logits_hbm.at[pl.ds(base + tok0, _CHUNK)])

        fire(0, 0, sem0)

        def do_chunk(c, _):
            even = (c % 2) == 0

            @pl.when(even)
            def _():
                step(c, 0, sem0, sem1)

            @pl.when(jnp.logical_not(even))
            def _():
                step(c, 1, sem1, sem0)

            return 0

        lax.fori_loop(0, n_chunks, do_chunk, 0)

    return kern


def _tc_bce_kernel(l_ref, p_ref, s_ref, n_ref):
    logits = l_ref[...]
    p = p_ref[...]
    code = ((p >> _CODE_SHIFT) & 1).astype(jnp.float32)
    m = ((p >> _MASK_SHIFT) & 1).astype(jnp.float32)
    el = (jnp.maximum(logits, 0.0) - logits * code
          + jnp.log1p(jnp.exp(-jnp.abs(logits))))
    s_ref[...] = jnp.sum(el * m).reshape(1, 1)
    n_ref[...] = jnp.sum(m).reshape(1, 1)


def kernel(embedding, target, fc, path_idx, path_codes, path_mask):
    emb = embedding.reshape(-1, embedding.shape[-1])
    tgt = target.reshape(-1).astype(jnp.int32)
    T = emb.shape[0]
    D = path_idx.shape[1]

    packed = (path_idx.astype(jnp.int32)
              | (path_codes.astype(jnp.int32) << _CODE_SHIFT)
              | (path_mask.astype(jnp.int32) << _MASK_SHIFT))
    packed = jnp.pad(packed, ((0, 0), (0, _DP - D)))

    logits, prows = _sc_logits_kernel(T, D)(packed, tgt, emb, fc)

    lr = logits.reshape(T * _DP // _H, _H)
    pr = prows.reshape(T * _DP // _H, _H)
    s, n = pl.pallas_call(
        _tc_bce_kernel,
        out_shape=(jax.ShapeDtypeStruct((1, 1), jnp.float32),
                   jax.ShapeDtypeStruct((1, 1), jnp.float32)),
    )(lr, pr)
    return s[0, 0] / n[0, 0]
